# Initial kernel scaffold; baseline (speedup 1.0000x reference)
import jax
import jax.numpy as jnp
from jax.experimental import pallas as pl


def kernel(emb, W_l1, b1, W_r1, W_l2, b2, W_r2, nodes, edge_index):
    raise NotImplementedError("write your pallas kernel here")


# speedup vs baseline: 5.4734x; 5.4734x over previous
import functools

import jax
import jax.numpy as jnp
from jax import lax
from jax.experimental import pallas as pl
from jax.experimental.pallas import tpu as pltpu, tpu_sc as plsc

N = 10000
D = 128
H = 128
C = 40
E = 320000
B = 1000

NC = 2
NS = 16
NPAD = 10240
ROWS_PER_TILE = NPAD // NS
TRASH = N
CHUNK = 128
CHUNKS_PER_TILE = 160
EP = NS * CHUNKS_PER_TILE * CHUNK
QP = 1024
Q_PER_TILE = QP // NS
DH = D // NC
GW = 64
GH = GW // NC

_mesh = plsc.VectorSubcoreMesh(core_axis_name="c", subcore_axis_name="s",
                               num_cores=NC, num_subcores=NS)


def _zero_2d(ref, rows, cols):
    def zrow(i, _):
        for c in range(cols // 16):
            ref[i, pl.ds(c * 16, 16)] = jnp.zeros((16,), jnp.float32)
        return 0
    lax.fori_loop(0, rows, zrow, 0)


def _load_and_offset_idx(srcw, dstw, sidx, didx, t, row_off):
    pltpu.sync_copy(srcw.at[pl.ds(t * CHUNKS_PER_TILE, CHUNKS_PER_TILE)], sidx)
    pltpu.sync_copy(dstw.at[pl.ds(t * CHUNKS_PER_TILE, CHUNKS_PER_TILE)], didx)
    off = jnp.full((16,), row_off, jnp.int32)

    def arow(i, _):
        for c in range(CHUNK // 16):
            sl = pl.ds(c * 16, 16)
            sidx[i, sl] = sidx[i, sl] + off
        return 0

    lax.fori_loop(0, CHUNKS_PER_TILE, arow, 0)


def _seg_sum_kernel(emb2_hbm, srcw, dstw, sum_out, cnt_out,
                    sidx, didx, bufA, bufB, ones, zc, tmp,
                    acc, cacc, semA, semB):
    c_id = lax.axis_index("c")
    t = lax.axis_index("s")
    base = t * ROWS_PER_TILE

    _zero_2d(tmp, 64, DH)
    for c in range(CHUNK // 16):
        ones[pl.ds(c * 16, 16)] = jnp.ones((16,), jnp.float32)
    for c in range(ROWS_PER_TILE // 16):
        zc[pl.ds(c * 16, 16)] = jnp.zeros((16,), jnp.float32)

    for kk in range(ROWS_PER_TILE // 64):
        pltpu.sync_copy(tmp, acc.at[pl.ds(base + kk * 64, 64)])
    pltpu.sync_copy(zc, cacc.at[pl.ds(base, ROWS_PER_TILE)])

    _load_and_offset_idx(srcw, dstw, sidx, didx, t, c_id * N)
    plsc.subcore_barrier()

    pltpu.async_copy(emb2_hbm.at[sidx.at[0]], bufA, semA)

    def step(j, _):
        c0 = 2 * j
        c1 = c0 + 1
        pltpu.make_async_copy(emb2_hbm.at[sidx.at[c0]], bufA, semA).wait()
        pltpu.async_copy(emb2_hbm.at[sidx.at[c1]], bufB, semB)
        pltpu.sync_copy(bufA, acc.at[didx.at[c0]], add=True)
        pltpu.sync_copy(ones, cacc.at[didx.at[c0]], add=True)
        pltpu.make_async_copy(emb2_hbm.at[sidx.at[c1]], bufB, semB).wait()

        @pl.when(j < CHUNKS_PER_TILE // 2 - 1)
        def _():
            pltpu.async_copy(emb2_hbm.at[sidx.at[c0 + 2]], bufA, semA)

        pltpu.sync_copy(bufB, acc.at[didx.at[c1]], add=True)
        pltpu.sync_copy(ones, cacc.at[didx.at[c1]], add=True)
        return 0

    lax.fori_loop(0, CHUNKS_PER_TILE // 2, step, 0)
    plsc.subcore_barrier()

    pltpu.sync_copy(acc.at[pl.ds(base, ROWS_PER_TILE)],
                    sum_out.at[c_id, pl.ds(base, ROWS_PER_TILE)])

    @pl.when(c_id == 0)
    def _():
        pltpu.sync_copy(cacc.at[pl.ds(base, ROWS_PER_TILE)],
                        cnt_out.at[pl.ds(base, ROWS_PER_TILE)])


_seg_sum = functools.partial(
    pl.kernel, _seg_sum_kernel, mesh=_mesh,
    compiler_params=pltpu.CompilerParams(use_tc_tiling_on_sc=False),
    out_type=[jax.ShapeDtypeStruct((NC, NPAD, DH), jnp.float32),
              jax.ShapeDtypeStruct((NPAD,), jnp.float32)],
    scratch_types=[
        pltpu.VMEM((CHUNKS_PER_TILE, CHUNK), jnp.int32),
        pltpu.VMEM((CHUNKS_PER_TILE, CHUNK), jnp.int32),
        pltpu.VMEM((CHUNK, DH), jnp.float32),
        pltpu.VMEM((CHUNK, DH), jnp.float32),
        pltpu.VMEM((CHUNK,), jnp.float32),
        pltpu.VMEM((ROWS_PER_TILE,), jnp.float32),
        pltpu.VMEM((64, DH), jnp.float32),
        pltpu.VMEM_SHARED((NPAD, DH), jnp.float32),
        pltpu.VMEM_SHARED((NPAD,), jnp.float32),
        pltpu.SemaphoreType.DMA,
        pltpu.SemaphoreType.DMA,
    ])()


def _dense_kernel(x_ref, sa_ref, sb_ref, cnt_ref, wl1a_ref, wl1b_ref,
                  wr1_ref, b1_ref, wl2a_ref, wl2b_ref, wr2a_ref, wr2b_ref,
                  b2a_ref, b2b_ref,
                  ga_ref, gb_ref, hra_ref, hrb_ref, ivb_ref):
    dn = (((1,), (1,)), ((), ()))
    hp = jax.lax.Precision.HIGHEST
    mm = lambda a, b: lax.dot_general(
        a, b, dn, preferred_element_type=jnp.float32, precision=hp)
    iv = 1.0 / jnp.maximum(cnt_ref[...], 1.0)
    h = mm(sa_ref[0] * iv, wl1a_ref[...])
    h = h + mm(sb_ref[0] * iv, wl1b_ref[...])
    h = h + mm(x_ref[...], wr1_ref[...])
    h = jnp.maximum(h + b1_ref[...], 0.0)
    ga_ref[...] = mm(h, wl2a_ref[...])
    gb_ref[...] = mm(h, wl2b_ref[...])
    hra_ref[...] = mm(h, wr2a_ref[...]) + b2a_ref[...]
    hrb_ref[...] = mm(h, wr2b_ref[...]) + b2b_ref[...]
    ivb_ref[...] = jnp.broadcast_to(iv, ivb_ref.shape)


def _dense(x, s2, cnt2d, wl1a, wl1b, wr1, b1, wl2a, wl2b, wr2a, wr2b,
           b2a, b2b):
    RB = 1280
    grid = (NPAD // RB,)
    full = lambda shape: pl.BlockSpec(shape, lambda i: tuple(0 for _ in shape))
    row = lambda cols: pl.BlockSpec((RB, cols), lambda i: (i, 0))
    return pl.pallas_call(
        _dense_kernel,
        grid=grid,
        in_specs=[
            row(D),
            pl.BlockSpec((1, RB, DH), lambda i: (0, i, 0)),
            pl.BlockSpec((1, RB, DH), lambda i: (1, i, 0)),
            row(1),
            full((H, DH)), full((H, DH)), full((H, D)), full((1, H)),
            full((GH, H)), full((GH, H)), full((GH, H)), full((GH, H)),
            full((1, GH)), full((1, GH)),
        ],
        out_specs=[row(GH), row(GH), row(GH), row(GH), row(D)],
        out_shape=[jax.ShapeDtypeStruct((NPAD, GH), jnp.float32),
                   jax.ShapeDtypeStruct((NPAD, GH), jnp.float32),
                   jax.ShapeDtypeStruct((NPAD, GH), jnp.float32),
                   jax.ShapeDtypeStruct((NPAD, GH), jnp.float32),
                   jax.ShapeDtypeStruct((NPAD, D), jnp.float32)],
    )(x, s2, s2, cnt2d, wl1a, wl1b, wr1, b1, wl2a, wl2b, wr2a, wr2b,
      b2a, b2b)


def _layer2_kernel(g2_hbm, srcw, dstw, hr2_hbm, ivb_hbm, nodes_hbm, out_hbm,
                   sidx, didx, bufA, bufB, tmp, nbuf, qsum, qhr, qiv, qout,
                   acc, semA, semB):
    c_id = lax.axis_index("c")
    t = lax.axis_index("s")
    base = t * ROWS_PER_TILE

    _zero_2d(tmp, 64, GH)
    for kk in range(ROWS_PER_TILE // 64):
        pltpu.sync_copy(tmp, acc.at[pl.ds(base + kk * 64, 64)])

    _load_and_offset_idx(srcw, dstw, sidx, didx, t, c_id * NPAD)
    plsc.subcore_barrier()

    pltpu.async_copy(g2_hbm.at[sidx.at[0]], bufA, semA)

    def step(j, _):
        c0 = 2 * j
        c1 = c0 + 1
        pltpu.make_async_copy(g2_hbm.at[sidx.at[c0]], bufA, semA).wait()
        pltpu.async_copy(g2_hbm.at[sidx.at[c1]], bufB, semB)
        pltpu.sync_copy(bufA, acc.at[didx.at[c0]], add=True)
        pltpu.make_async_copy(g2_hbm.at[sidx.at[c1]], bufB, semB).wait()

        @pl.when(j < CHUNKS_PER_TILE // 2 - 1)
        def _():
            pltpu.async_copy(g2_hbm.at[sidx.at[c0 + 2]], bufA, semA)

        pltpu.sync_copy(bufB, acc.at[didx.at[c1]], add=True)
        return 0

    lax.fori_loop(0, CHUNKS_PER_TILE // 2, step, 0)
    plsc.subcore_barrier()

    pltpu.sync_copy(nodes_hbm.at[pl.ds(t * Q_PER_TILE, Q_PER_TILE)], nbuf)
    off = jnp.full((16,), c_id * NPAD, jnp.int32)
    for c in range(Q_PER_TILE // 16):
        sl = pl.ds(c * 16, 16)
        nbuf[sl] = nbuf[sl] + off
    pltpu.async_copy(hr2_hbm.at[nbuf], qhr, semA).wait()
    for c in range(Q_PER_TILE // 16):
        sl = pl.ds(c * 16, 16)
        nbuf[sl] = nbuf[sl] - off
    pltpu.async_copy(acc.at[nbuf], qsum, semA).wait()
    pltpu.async_copy(ivb_hbm.at[nbuf], qiv, semA).wait()

    def qrow(q, _):
        iv = qiv[q, pl.ds(0, 16)]
        for c in range(GH // 16):
            sl = pl.ds(c * 16, 16)
            qout[q, sl] = qsum[q, sl] * iv + qhr[q, sl]
        return 0

    lax.fori_loop(0, Q_PER_TILE, qrow, 0)
    pltpu.sync_copy(qout, out_hbm.at[c_id, pl.ds(t * Q_PER_TILE, Q_PER_TILE)])


_layer2 = functools.partial(
    pl.kernel, _layer2_kernel, mesh=_mesh,
    compiler_params=pltpu.CompilerParams(use_tc_tiling_on_sc=False),
    out_type=jax.ShapeDtypeStruct((NC, QP, GH), jnp.float32),
    scratch_types=[
        pltpu.VMEM((CHUNKS_PER_TILE, CHUNK), jnp.int32),
        pltpu.VMEM((CHUNKS_PER_TILE, CHUNK), jnp.int32),
        pltpu.VMEM((CHUNK, GH), jnp.float32),
        pltpu.VMEM((CHUNK, GH), jnp.float32),
        pltpu.VMEM((64, GH), jnp.float32),
        pltpu.VMEM((Q_PER_TILE,), jnp.int32),
        pltpu.VMEM((Q_PER_TILE, GH), jnp.float32),
        pltpu.VMEM((Q_PER_TILE, GH), jnp.float32),
        pltpu.VMEM((Q_PER_TILE, D), jnp.float32),
        pltpu.VMEM((Q_PER_TILE, GH), jnp.float32),
        pltpu.VMEM_SHARED((NPAD, GH), jnp.float32),
        pltpu.SemaphoreType.DMA,
        pltpu.SemaphoreType.DMA,
    ])()


def kernel(emb, W_l1, b1, W_r1, W_l2, b2, W_r2, nodes, edge_index):
    src = edge_index[0].astype(jnp.int32)
    dst = edge_index[1].astype(jnp.int32)
    pad = EP - E
    srcw = jnp.concatenate(
        [src, jnp.zeros((pad,), jnp.int32)]).reshape(EP // CHUNK, CHUNK)
    dstw = jnp.concatenate(
        [dst, jnp.full((pad,), TRASH, jnp.int32)]).reshape(EP // CHUNK, CHUNK)
    nodesp = jnp.concatenate(
        [nodes.astype(jnp.int32), jnp.zeros((QP - B,), jnp.int32)])

    emb2 = jnp.concatenate([emb[:, :DH], emb[:, DH:]], axis=0)
    sum2, cnt = _seg_sum(emb2, srcw, dstw)

    embp = jnp.pad(emb, ((0, NPAD - N), (0, 0)))
    wl2p = jnp.pad(W_l2, ((0, GW - C), (0, 0)))
    wr2p = jnp.pad(W_r2, ((0, GW - C), (0, 0)))
    b2p = jnp.pad(b2, (0, GW - C))
    ga, gb, hra, hrb, ivb = _dense(
        embp, sum2, cnt.reshape(NPAD, 1),
        W_l1[:, :DH], W_l1[:, DH:], W_r1, b1.reshape(1, H),
        wl2p[:GH], wl2p[GH:], wr2p[:GH], wr2p[GH:],
        b2p[:GH].reshape(1, GH), b2p[GH:].reshape(1, GH))

    g2 = jnp.concatenate([ga, gb], axis=0)
    hr2 = jnp.concatenate([hra, hrb], axis=0)
    outq = _layer2(g2, srcw, dstw, hr2, ivb, nodesp)
    return jnp.concatenate([outq[0], outq[1]], axis=1)[:B, :C]


# speedup vs baseline: 6.2139x; 1.1353x over previous
import functools

import jax
import jax.numpy as jnp
from jax import lax
from jax.experimental import pallas as pl
from jax.experimental.pallas import tpu as pltpu, tpu_sc as plsc

N = 10000
D = 128
H = 128
C = 40
E = 320000
B = 1000

NC = 2
NS = 16
NPAD = 10240
ROWS_PER_TILE = NPAD // NS
TRASH = N
CHUNK = 128
CHUNKS_PER_TILE = 160
EP = NS * CHUNKS_PER_TILE * CHUNK
QP = 1024
Q_PER_TILE = QP // NS
DH = D // NC
GW = 64
GH = GW // NC
RING = 4

_mesh = plsc.VectorSubcoreMesh(core_axis_name="c", subcore_axis_name="s",
                               num_cores=NC, num_subcores=NS)


def _zero_2d(ref, rows, cols):
    def zrow(i, _):
        for c in range(cols // 16):
            ref[i, pl.ds(c * 16, 16)] = jnp.zeros((16,), jnp.float32)
        return 0
    lax.fori_loop(0, rows, zrow, 0)


def _load_and_offset_idx(srcw, dstw, sidx, didx, t, row_off):
    pltpu.sync_copy(srcw.at[pl.ds(t * CHUNKS_PER_TILE, CHUNKS_PER_TILE)], sidx)
    pltpu.sync_copy(dstw.at[pl.ds(t * CHUNKS_PER_TILE, CHUNKS_PER_TILE)], didx)
    off = jnp.full((16,), row_off, jnp.int32)

    def arow(i, _):
        for c in range(CHUNK // 16):
            sl = pl.ds(c * 16, 16)
            sidx[i, sl] = sidx[i, sl] + off
        return 0

    lax.fori_loop(0, CHUNKS_PER_TILE, arow, 0)


def _ring_pipeline(table, sidx, didx, acc, bufs, gsems, ssems, extra=None):
    for k in range(2):
        pltpu.async_copy(table.at[sidx.at[k]], bufs[k], gsems[k])

    def macro(m, _):
        for k in range(RING):
            j = RING * m + k
            pltpu.make_async_copy(table.at[sidx.at[j]], bufs[k],
                                  gsems[k]).wait()
            pltpu.async_copy(bufs[k], acc.at[didx.at[j]], ssems[k],
                             add=True)
            if extra is not None:
                extra(k, j)
            bn = (k + 2) % RING
            jn = j + 2

            @pl.when(jn < CHUNKS_PER_TILE)
            def _():
                @pl.when(j >= 2)
                def _():
                    pltpu.make_async_copy(bufs[bn], acc.at[didx.at[0]],
                                          ssems[bn]).wait()

                pltpu.async_copy(table.at[sidx.at[jn]], bufs[bn], gsems[bn])

        return 0

    lax.fori_loop(0, CHUNKS_PER_TILE // RING, macro, 0)
    for k in (2, 3):
        pltpu.make_async_copy(bufs[k], acc.at[didx.at[0]], ssems[k]).wait()


def _seg_sum_kernel(emb2_hbm, srcw, dstw, sum_out, cnt_out,
                    sidx, didx, b0, b1, b2, b3, ones, zc, tmp,
                    acc, cacc, g0, g1, g2, g3, s0, s1, s2, s3, semC):
    c_id = lax.axis_index("c")
    t = lax.axis_index("s")
    base = t * ROWS_PER_TILE

    _zero_2d(tmp, 64, DH)
    for c in range(CHUNK // 16):
        ones[pl.ds(c * 16, 16)] = jnp.ones((16,), jnp.float32)
    for c in range(ROWS_PER_TILE // 16):
        zc[pl.ds(c * 16, 16)] = jnp.zeros((16,), jnp.float32)

    for kk in range(ROWS_PER_TILE // 64):
        pltpu.sync_copy(tmp, acc.at[pl.ds(base + kk * 64, 64)])
    pltpu.sync_copy(zc, cacc.at[pl.ds(base, ROWS_PER_TILE)])

    _load_and_offset_idx(srcw, dstw, sidx, didx, t, c_id * N)
    plsc.subcore_barrier()

    def cnt_extra(k, j):
        @pl.when(c_id == (k % 2))
        def _():
            pltpu.async_copy(ones, cacc.at[didx.at[j]], semC, add=True)

    _ring_pipeline(emb2_hbm, sidx, didx, acc, (b0, b1, b2, b3),
                   (g0, g1, g2, g3), (s0, s1, s2, s3), extra=cnt_extra)

    def drainC(i, _):
        pltpu.make_async_copy(ones, cacc.at[didx.at[0]], semC).wait()
        return 0

    lax.fori_loop(0, CHUNKS_PER_TILE // 2, drainC, 0)
    plsc.subcore_barrier()

    pltpu.sync_copy(acc.at[pl.ds(base, ROWS_PER_TILE)],
                    sum_out.at[c_id, pl.ds(base, ROWS_PER_TILE)])
    pltpu.sync_copy(cacc.at[pl.ds(base, ROWS_PER_TILE)],
                    cnt_out.at[c_id, pl.ds(base, ROWS_PER_TILE)])


_seg_sum = functools.partial(
    pl.kernel, _seg_sum_kernel, mesh=_mesh,
    compiler_params=pltpu.CompilerParams(use_tc_tiling_on_sc=False),
    out_type=[jax.ShapeDtypeStruct((NC, NPAD, DH), jnp.float32),
              jax.ShapeDtypeStruct((NC, NPAD), jnp.float32)],
    scratch_types=[
        pltpu.VMEM((CHUNKS_PER_TILE, CHUNK), jnp.int32),
        pltpu.VMEM((CHUNKS_PER_TILE, CHUNK), jnp.int32),
        pltpu.VMEM((CHUNK, DH), jnp.float32),
        pltpu.VMEM((CHUNK, DH), jnp.float32),
        pltpu.VMEM((CHUNK, DH), jnp.float32),
        pltpu.VMEM((CHUNK, DH), jnp.float32),
        pltpu.VMEM((CHUNK,), jnp.float32),
        pltpu.VMEM((ROWS_PER_TILE,), jnp.float32),
        pltpu.VMEM((64, DH), jnp.float32),
        pltpu.VMEM_SHARED((NPAD, DH), jnp.float32),
        pltpu.VMEM_SHARED((NPAD,), jnp.float32),
        pltpu.SemaphoreType.DMA,
        pltpu.SemaphoreType.DMA,
        pltpu.SemaphoreType.DMA,
        pltpu.SemaphoreType.DMA,
        pltpu.SemaphoreType.DMA,
        pltpu.SemaphoreType.DMA,
        pltpu.SemaphoreType.DMA,
        pltpu.SemaphoreType.DMA,
        pltpu.SemaphoreType.DMA,
    ])()


def _dense_kernel(x_ref, sa_ref, sb_ref, ca_ref, cb_ref, wl1a_ref, wl1b_ref,
                  wr1_ref, b1_ref, wl2a_ref, wl2b_ref, wr2a_ref, wr2b_ref,
                  b2a_ref, b2b_ref,
                  ga_ref, gb_ref, hra_ref, hrb_ref, ivb_ref):
    dn = (((1,), (1,)), ((), ()))
    hp = jax.lax.Precision.HIGHEST
    mm = lambda a, b: lax.dot_general(
        a, b, dn, preferred_element_type=jnp.float32, precision=hp)
    cnt = ca_ref[0] + cb_ref[0]
    iv = 1.0 / jnp.maximum(cnt, 1.0)
    h = mm(sa_ref[0] * iv, wl1a_ref[...])
    h = h + mm(sb_ref[0] * iv, wl1b_ref[...])
    h = h + mm(x_ref[...], wr1_ref[...])
    h = jnp.maximum(h + b1_ref[...], 0.0)
    ga_ref[...] = mm(h, wl2a_ref[...])
    gb_ref[...] = mm(h, wl2b_ref[...])
    hra_ref[...] = mm(h, wr2a_ref[...]) + b2a_ref[...]
    hrb_ref[...] = mm(h, wr2b_ref[...]) + b2b_ref[...]
    ivb_ref[...] = jnp.broadcast_to(iv, ivb_ref.shape)


def _dense(x, s2, cnt2, wl1a, wl1b, wr1, b1, wl2a, wl2b, wr2a, wr2b,
           b2a, b2b):
    RB = 1280
    grid = (NPAD // RB,)
    full = lambda shape: pl.BlockSpec(shape, lambda i: tuple(0 for _ in shape))
    row = lambda cols: pl.BlockSpec((RB, cols), lambda i: (i, 0))
    halfrow = lambda c, cols: pl.BlockSpec((1, RB, cols),
                                           lambda i, c=c: (c, i, 0))
    return pl.pallas_call(
        _dense_kernel,
        grid=grid,
        in_specs=[
            row(D),
            halfrow(0, DH), halfrow(1, DH),
            halfrow(0, 1), halfrow(1, 1),
            full((H, DH)), full((H, DH)), full((H, D)), full((1, H)),
            full((GH, H)), full((GH, H)), full((GH, H)), full((GH, H)),
            full((1, GH)), full((1, GH)),
        ],
        out_specs=[row(GH), row(GH), row(GH), row(GH), row(D)],
        out_shape=[jax.ShapeDtypeStruct((NPAD, GH), jnp.float32),
                   jax.ShapeDtypeStruct((NPAD, GH), jnp.float32),
                   jax.ShapeDtypeStruct((NPAD, GH), jnp.float32),
                   jax.ShapeDtypeStruct((NPAD, GH), jnp.float32),
                   jax.ShapeDtypeStruct((NPAD, D), jnp.float32)],
    )(x, s2, s2, cnt2, cnt2, wl1a, wl1b, wr1, b1, wl2a, wl2b, wr2a, wr2b,
      b2a, b2b)


def _layer2_kernel(g2_hbm, srcw, dstw, hr2_hbm, ivb_hbm, nodes_hbm, out_hbm,
                   sidx, didx, b0, b1, b2, b3, tmp, nbuf, qsum, qhr, qiv,
                   qout, acc, g0, g1, g2, g3, s0, s1, s2, s3):
    c_id = lax.axis_index("c")
    t = lax.axis_index("s")
    base = t * ROWS_PER_TILE

    _zero_2d(tmp, 64, GH)
    for kk in range(ROWS_PER_TILE // 64):
        pltpu.sync_copy(tmp, acc.at[pl.ds(base + kk * 64, 64)])

    _load_and_offset_idx(srcw, dstw, sidx, didx, t, c_id * NPAD)
    plsc.subcore_barrier()

    _ring_pipeline(g2_hbm, sidx, didx, acc, (b0, b1, b2, b3),
                   (g0, g1, g2, g3), (s0, s1, s2, s3))
    plsc.subcore_barrier()

    pltpu.sync_copy(nodes_hbm.at[pl.ds(t * Q_PER_TILE, Q_PER_TILE)], nbuf)
    off = jnp.full((16,), c_id * NPAD, jnp.int32)
    for c in range(Q_PER_TILE // 16):
        sl = pl.ds(c * 16, 16)
        nbuf[sl] = nbuf[sl] + off
    pltpu.async_copy(hr2_hbm.at[nbuf], qhr, g0).wait()
    for c in range(Q_PER_TILE // 16):
        sl = pl.ds(c * 16, 16)
        nbuf[sl] = nbuf[sl] - off
    pltpu.async_copy(acc.at[nbuf], qsum, g0).wait()
    pltpu.async_copy(ivb_hbm.at[nbuf], qiv, g0).wait()

    def qrow(q, _):
        iv = qiv[q, pl.ds(0, 16)]
        for c in range(GH // 16):
            sl = pl.ds(c * 16, 16)
            qout[q, sl] = qsum[q, sl] * iv + qhr[q, sl]
        return 0

    lax.fori_loop(0, Q_PER_TILE, qrow, 0)
    pltpu.sync_copy(qout, out_hbm.at[c_id, pl.ds(t * Q_PER_TILE, Q_PER_TILE)])


_layer2 = functools.partial(
    pl.kernel, _layer2_kernel, mesh=_mesh,
    compiler_params=pltpu.CompilerParams(use_tc_tiling_on_sc=False),
    out_type=jax.ShapeDtypeStruct((NC, QP, GH), jnp.float32),
    scratch_types=[
        pltpu.VMEM((CHUNKS_PER_TILE, CHUNK), jnp.int32),
        pltpu.VMEM((CHUNKS_PER_TILE, CHUNK), jnp.int32),
        pltpu.VMEM((CHUNK, GH), jnp.float32),
        pltpu.VMEM((CHUNK, GH), jnp.float32),
        pltpu.VMEM((CHUNK, GH), jnp.float32),
        pltpu.VMEM((CHUNK, GH), jnp.float32),
        pltpu.VMEM((64, GH), jnp.float32),
        pltpu.VMEM((Q_PER_TILE,), jnp.int32),
        pltpu.VMEM((Q_PER_TILE, GH), jnp.float32),
        pltpu.VMEM((Q_PER_TILE, GH), jnp.float32),
        pltpu.VMEM((Q_PER_TILE, D), jnp.float32),
        pltpu.VMEM((Q_PER_TILE, GH), jnp.float32),
        pltpu.VMEM_SHARED((NPAD, GH), jnp.float32),
        pltpu.SemaphoreType.DMA,
        pltpu.SemaphoreType.DMA,
        pltpu.SemaphoreType.DMA,
        pltpu.SemaphoreType.DMA,
        pltpu.SemaphoreType.DMA,
        pltpu.SemaphoreType.DMA,
        pltpu.SemaphoreType.DMA,
        pltpu.SemaphoreType.DMA,
    ])()


def kernel(emb, W_l1, b1, W_r1, W_l2, b2, W_r2, nodes, edge_index):
    src = edge_index[0].astype(jnp.int32)
    dst = edge_index[1].astype(jnp.int32)
    pad = EP - E
    srcw = jnp.concatenate(
        [src, jnp.zeros((pad,), jnp.int32)]).reshape(EP // CHUNK, CHUNK)
    dstw = jnp.concatenate(
        [dst, jnp.full((pad,), TRASH, jnp.int32)]).reshape(EP // CHUNK, CHUNK)
    nodesp = jnp.concatenate(
        [nodes.astype(jnp.int32), jnp.zeros((QP - B,), jnp.int32)])

    emb2 = jnp.concatenate([emb[:, :DH], emb[:, DH:]], axis=0)
    sum2, cnt2 = _seg_sum(emb2, srcw, dstw)

    embp = jnp.pad(emb, ((0, NPAD - N), (0, 0)))
    wl2p = jnp.pad(W_l2, ((0, GW - C), (0, 0)))
    wr2p = jnp.pad(W_r2, ((0, GW - C), (0, 0)))
    b2p = jnp.pad(b2, (0, GW - C))
    ga, gb, hra, hrb, ivb = _dense(
        embp, sum2, cnt2.reshape(NC, NPAD, 1),
        W_l1[:, :DH], W_l1[:, DH:], W_r1, b1.reshape(1, H),
        wl2p[:GH], wl2p[GH:], wr2p[:GH], wr2p[GH:],
        b2p[:GH].reshape(1, GH), b2p[GH:].reshape(1, GH))

    g2 = jnp.concatenate([ga, gb], axis=0)
    hr2 = jnp.concatenate([hra, hrb], axis=0)
    outq = _layer2(g2, srcw, dstw, hr2, ivb, nodesp)
    return jnp.concatenate([outq[0], outq[1]], axis=1)[:B, :C]
